# Initial kernel scaffold; baseline (speedup 1.0000x reference)
import jax
import jax.numpy as jnp
from jax.experimental import pallas as pl


def kernel(enhanced_posts_embeddings, selected_reasoning_embeddings, llm_embeddings, gate_W, gate_b, U_W, U_b, V_W, V_b, rand_u):
    raise NotImplementedError("write your pallas kernel here")


# speedup vs baseline: 11.3855x; 11.3855x over previous
import jax
import jax.numpy as jnp
from jax.experimental import pallas as pl
from jax.experimental.pallas import tpu as pltpu

B = 8192
D = 384
H = 64
R = 8
NL = 64
TEMP = 1.0
AUX = 0.05

BLK = 1024
NBLK = B // BLK

_HI = jax.lax.Precision.HIGHEST


def _v_kernel(llm_ref, vwt_ref, vb_ref, out_ref):
    vm = jnp.dot(llm_ref[:], vwt_ref[:], preferred_element_type=jnp.float32,
                 precision=_HI) + vb_ref[:]
    for r in range(R):
        sl = slice(r * H, (r + 1) * H)
        vr = vm[:, sl]
        n = jnp.sqrt(jnp.sum(vr * vr, axis=1, keepdims=True))
        out_ref[:, sl] = vr / jnp.maximum(n, 1e-12)


def _main_kernel(xp_ref, xr_ref, gwt_ref, gb_ref, uwt_ref, ub_ref, v_ref,
                 rand_ref, sel_ref, logp_ref, aux_ref, psum_ref, msum_ref):
    i = pl.program_id(0)
    xp = xp_ref[:]
    xr = xr_ref[:]
    gwt = gwt_ref[:]
    uwt = uwt_ref[:]

    logits = (jnp.dot(xp, gwt[:D], preferred_element_type=jnp.float32, precision=_HI)
              + jnp.dot(xr, gwt[D:], preferred_element_type=jnp.float32, precision=_HI)
              + gb_ref[:])

    u_all = (jnp.dot(xp, uwt[:D], preferred_element_type=jnp.float32, precision=_HI)
             + jnp.dot(xr, uwt[D:], preferred_element_type=jnp.float32, precision=_HI)
             + ub_ref[:])

    m = jnp.max(logits, axis=1, keepdims=True)
    e = jnp.exp(logits - m)
    probs = e / jnp.sum(e, axis=1, keepdims=True)

    iota8 = jax.lax.broadcasted_iota(jnp.int32, (BLK, R), 1)
    m1 = jnp.max(logits, axis=1, keepdims=True)
    i1 = jnp.min(jnp.where(logits == m1, iota8, R), axis=1, keepdims=True)
    neg = jnp.where(iota8 == i1, -jnp.inf, logits)
    m2 = jnp.max(neg, axis=1, keepdims=True)
    i2 = jnp.min(jnp.where(neg == m2, iota8, R), axis=1, keepdims=True)

    ew = jnp.exp(m2 - m1)
    denom = 1.0 + ew
    w1 = 1.0 / denom
    w2 = ew / denom
    wvec = (jnp.where(iota8 == i1, w1, 0.0) + jnp.where(iota8 == i2, w2, 0.0))
    maskvec = (jnp.where(iota8 == i1, 1.0, 0.0) + jnp.where(iota8 == i2, 1.0, 0.0))

    @pl.when(i == 0)
    def _():
        psum_ref[:] = jnp.zeros_like(psum_ref)
        msum_ref[:] = jnp.zeros_like(msum_ref)

    psum_ref[:] += jnp.sum(probs, axis=0, keepdims=True)
    msum_ref[:] += jnp.sum(maskvec, axis=0, keepdims=True)

    llm_probs = jnp.zeros((BLK, NL), dtype=jnp.float32)
    for r in range(R):
        sl = slice(r * H, (r + 1) * H)
        ur = u_all[:, sl]
        n = jnp.sqrt(jnp.sum(ur * ur, axis=1, keepdims=True))
        urn = ur / jnp.maximum(n, 1e-12)
        vr = v_ref[:, sl]
        sc = jax.lax.dot_general(urn, vr, (((1,), (1,)), ((), ())),
                                 preferred_element_type=jnp.float32,
                                 precision=_HI) * (1.0 / TEMP)
        ms = jnp.max(sc, axis=1, keepdims=True)
        es = jnp.exp(sc - ms)
        ro = es / jnp.sum(es, axis=1, keepdims=True)
        llm_probs += wvec[:, r:r + 1] * ro

    io_r = jax.lax.broadcasted_iota(jnp.int32, (NL, NL), 0)
    io_c = jax.lax.broadcasted_iota(jnp.int32, (NL, NL), 1)
    tri = (io_r <= io_c).astype(jnp.float32)
    csum = jnp.dot(llm_probs, tri, preferred_element_type=jnp.float32,
                   precision=_HI)
    iota64 = jax.lax.broadcasted_iota(jnp.int32, (BLK, NL), 1)
    gt = csum > rand_ref[:]
    idx = jnp.min(jnp.where(gt, iota64, NL), axis=1, keepdims=True)
    idx = jnp.where(idx == NL, 0, idx)
    sel_ref[:] = idx

    pick = jnp.sum(jnp.where(iota64 == idx, llm_probs, 0.0), axis=1,
                   keepdims=True)
    logp_ref[:] = jnp.log(pick)

    @pl.when(i == NBLK - 1)
    def _():
        aux_ref[:] = (jnp.sum(psum_ref[:] * msum_ref[:], axis=(0, 1),
                              keepdims=True) * (R * AUX / (B * B)))


def kernel(enhanced_posts_embeddings, selected_reasoning_embeddings,
           llm_embeddings, gate_W, gate_b, U_W, U_b, V_W, V_b, rand_u):
    gwt = gate_W.T
    gb = gate_b.reshape(1, R)
    uwt = U_W.reshape(R * H, 2 * D).T
    ub = U_b.reshape(1, R * H)
    vwt = V_W.reshape(R * H, D).T
    vb = V_b.reshape(1, R * H)

    v_norm = pl.pallas_call(
        _v_kernel,
        out_shape=jax.ShapeDtypeStruct((NL, R * H), jnp.float32),
    )(llm_embeddings, vwt, vb)

    blk = lambda *shape: pl.BlockSpec(shape, lambda i: (0,) * len(shape))
    sel, logp, aux, _, _ = pl.pallas_call(
        _main_kernel,
        grid=(NBLK,),
        in_specs=[
            pl.BlockSpec((BLK, D), lambda i: (i, 0)),
            pl.BlockSpec((BLK, D), lambda i: (i, 0)),
            blk(2 * D, R),
            blk(1, R),
            blk(2 * D, R * H),
            blk(1, R * H),
            blk(NL, R * H),
            pl.BlockSpec((BLK, 1), lambda i: (i, 0)),
        ],
        out_specs=[
            pl.BlockSpec((BLK, 1), lambda i: (i, 0)),
            pl.BlockSpec((BLK, 1), lambda i: (i, 0)),
            blk(1, 1),
            blk(1, R),
            blk(1, R),
        ],
        out_shape=[
            jax.ShapeDtypeStruct((B, 1), jnp.int32),
            jax.ShapeDtypeStruct((B, 1), jnp.float32),
            jax.ShapeDtypeStruct((1, 1), jnp.float32),
            jax.ShapeDtypeStruct((1, R), jnp.float32),
            jax.ShapeDtypeStruct((1, R), jnp.float32),
        ],
        compiler_params=pltpu.CompilerParams(
            dimension_semantics=("arbitrary",),
        ),
    )(enhanced_posts_embeddings, selected_reasoning_embeddings, gwt, gb,
      uwt, ub, v_norm, rand_u)

    return sel.reshape(B), logp, aux.reshape(())


# speedup vs baseline: 12.3870x; 1.0880x over previous
import jax
import jax.numpy as jnp
from jax.experimental import pallas as pl
from jax.experimental.pallas import tpu as pltpu

B = 8192
D = 384
H = 64
R = 8
NL = 64
TEMP = 1.0
AUX = 0.05

BLK = 1024
NBLK = B // BLK

_HI = jax.lax.Precision.HIGHEST


def _v_kernel(llm_ref, vwt_ref, vb_ref, out_ref):
    vm = jnp.dot(llm_ref[:], vwt_ref[:], preferred_element_type=jnp.float32,
                 precision=_HI) + vb_ref[:]
    for r in range(R):
        sl = slice(r * H, (r + 1) * H)
        vr = vm[:, sl]
        n = jnp.sqrt(jnp.sum(vr * vr, axis=1, keepdims=True))
        out_ref[:, sl] = vr / jnp.maximum(n, 1e-12)


def _aux_kernel(psum_ref, msum_ref, aux_ref):
    p = jnp.sum(psum_ref[:], axis=0, keepdims=True)
    m = jnp.sum(msum_ref[:], axis=0, keepdims=True)
    aux_ref[:] = jnp.sum(p * m, axis=(0, 1), keepdims=True) * (R * AUX / (B * B))


def _main_kernel(xp_ref, xr_ref, gwt_ref, gb_ref, uwt_ref, ub_ref, v_ref,
                 rand_ref, sel_ref, logp_ref, psum_ref, msum_ref):
    xp = xp_ref[:]
    xr = xr_ref[:]
    gwt = gwt_ref[:]
    uwt = uwt_ref[:]

    logits = (jnp.dot(xp, gwt[:D], preferred_element_type=jnp.float32, precision=_HI)
              + jnp.dot(xr, gwt[D:], preferred_element_type=jnp.float32, precision=_HI)
              + gb_ref[:])

    u_all = (jnp.dot(xp, uwt[:D], preferred_element_type=jnp.float32, precision=_HI)
             + jnp.dot(xr, uwt[D:], preferred_element_type=jnp.float32, precision=_HI)
             + ub_ref[:])

    m = jnp.max(logits, axis=1, keepdims=True)
    e = jnp.exp(logits - m)
    probs = e / jnp.sum(e, axis=1, keepdims=True)

    iota8 = jax.lax.broadcasted_iota(jnp.int32, (BLK, R), 1)
    m1 = jnp.max(logits, axis=1, keepdims=True)
    i1 = jnp.min(jnp.where(logits == m1, iota8, R), axis=1, keepdims=True)
    neg = jnp.where(iota8 == i1, -jnp.inf, logits)
    m2 = jnp.max(neg, axis=1, keepdims=True)
    i2 = jnp.min(jnp.where(neg == m2, iota8, R), axis=1, keepdims=True)

    ew = jnp.exp(m2 - m1)
    denom = 1.0 + ew
    w1 = 1.0 / denom
    w2 = ew / denom
    maskvec = (jnp.where(iota8 == i1, 1.0, 0.0) + jnp.where(iota8 == i2, 1.0, 0.0))

    psum_ref[:] = jnp.sum(probs, axis=0, keepdims=True).reshape(1, 1, R)
    msum_ref[:] = jnp.sum(maskvec, axis=0, keepdims=True).reshape(1, 1, R)

    sc1 = jnp.zeros((BLK, NL), dtype=jnp.float32)
    sc2 = jnp.zeros((BLK, NL), dtype=jnp.float32)
    for r in range(R):
        sl = slice(r * H, (r + 1) * H)
        ur = u_all[:, sl]
        n = jnp.sqrt(jnp.sum(ur * ur, axis=1, keepdims=True))
        urn = ur / jnp.maximum(n, 1e-12)
        vr = v_ref[:, sl]
        sc = jax.lax.dot_general(urn, vr, (((1,), (1,)), ((), ())),
                                 preferred_element_type=jnp.float32,
                                 precision=_HI) * (1.0 / TEMP)
        sc1 += jnp.where(i1 == r, sc, 0.0)
        sc2 += jnp.where(i2 == r, sc, 0.0)

    llm_probs = jnp.zeros((BLK, NL), dtype=jnp.float32)
    for sck, wk in ((sc1, w1), (sc2, w2)):
        ms = jnp.max(sck, axis=1, keepdims=True)
        es = jnp.exp(sck - ms)
        ro = es / jnp.sum(es, axis=1, keepdims=True)
        llm_probs += wk * ro

    io_r = jax.lax.broadcasted_iota(jnp.int32, (NL, NL), 0)
    io_c = jax.lax.broadcasted_iota(jnp.int32, (NL, NL), 1)
    tri = (io_r <= io_c).astype(jnp.float32)
    csum = jnp.dot(llm_probs, tri, preferred_element_type=jnp.float32,
                   precision=_HI)
    iota64 = jax.lax.broadcasted_iota(jnp.int32, (BLK, NL), 1)
    gt = csum > rand_ref[:]
    idx = jnp.min(jnp.where(gt, iota64, NL), axis=1, keepdims=True)
    idx = jnp.where(idx == NL, 0, idx)
    sel_ref[:] = idx

    pick = jnp.sum(jnp.where(iota64 == idx, llm_probs, 0.0), axis=1,
                   keepdims=True)
    logp_ref[:] = jnp.log(pick)


def kernel(enhanced_posts_embeddings, selected_reasoning_embeddings,
           llm_embeddings, gate_W, gate_b, U_W, U_b, V_W, V_b, rand_u):
    gwt = gate_W.T
    gb = gate_b.reshape(1, R)
    uwt = U_W.reshape(R * H, 2 * D).T
    ub = U_b.reshape(1, R * H)
    vwt = V_W.reshape(R * H, D).T
    vb = V_b.reshape(1, R * H)

    v_norm = pl.pallas_call(
        _v_kernel,
        out_shape=jax.ShapeDtypeStruct((NL, R * H), jnp.float32),
    )(llm_embeddings, vwt, vb)

    blk = lambda *shape: pl.BlockSpec(shape, lambda i: (0,) * len(shape))
    sel, logp, psum, msum = pl.pallas_call(
        _main_kernel,
        grid=(NBLK,),
        in_specs=[
            pl.BlockSpec((BLK, D), lambda i: (i, 0)),
            pl.BlockSpec((BLK, D), lambda i: (i, 0)),
            blk(2 * D, R),
            blk(1, R),
            blk(2 * D, R * H),
            blk(1, R * H),
            blk(NL, R * H),
            pl.BlockSpec((BLK, 1), lambda i: (i, 0)),
        ],
        out_specs=[
            pl.BlockSpec((BLK, 1), lambda i: (i, 0)),
            pl.BlockSpec((BLK, 1), lambda i: (i, 0)),
            pl.BlockSpec((1, 1, R), lambda i: (i, 0, 0)),
            pl.BlockSpec((1, 1, R), lambda i: (i, 0, 0)),
        ],
        out_shape=[
            jax.ShapeDtypeStruct((B, 1), jnp.int32),
            jax.ShapeDtypeStruct((B, 1), jnp.float32),
            jax.ShapeDtypeStruct((NBLK, 1, R), jnp.float32),
            jax.ShapeDtypeStruct((NBLK, 1, R), jnp.float32),
        ],
        compiler_params=pltpu.CompilerParams(
            dimension_semantics=("parallel",),
        ),
    )(enhanced_posts_embeddings, selected_reasoning_embeddings, gwt, gb,
      uwt, ub, v_norm, rand_u)

    aux = pl.pallas_call(
        _aux_kernel,
        out_shape=jax.ShapeDtypeStruct((1, 1), jnp.float32),
    )(psum.reshape(NBLK, R), msum.reshape(NBLK, R))

    return sel.reshape(B), logp, aux.reshape(())


# speedup vs baseline: 16.6806x; 1.3466x over previous
import jax
import jax.numpy as jnp
from jax.experimental import pallas as pl
from jax.experimental.pallas import tpu as pltpu

B = 8192
D = 384
H = 64
R = 8
NL = 64
TEMP = 1.0
AUX = 0.05

BLK = 1024
NBLK = B // BLK

_HI = jax.lax.Precision.HIGHEST


def _v_kernel(llm_ref, vwt_ref, vb_ref, out_ref):
    vm = jnp.dot(llm_ref[:], vwt_ref[:], preferred_element_type=jnp.float32,
                 precision=_HI) + vb_ref[:]
    for r in range(R):
        sl = slice(r * H, (r + 1) * H)
        vr = vm[:, sl]
        n = jnp.sqrt(jnp.sum(vr * vr, axis=1, keepdims=True))
        out_ref[:, sl] = vr / jnp.maximum(n, 1e-12)


def _aux_kernel(psum_ref, msum_ref, aux_ref):
    p = jnp.sum(psum_ref[:], axis=0, keepdims=True)
    m = jnp.sum(msum_ref[:], axis=0, keepdims=True)
    aux_ref[:] = jnp.sum(p * m, axis=(0, 1), keepdims=True) * (R * AUX / (B * B))


def _main_kernel(xp_ref, xr_ref, uwg_ref, ubg_ref, v_ref,
                 rand_ref, sel_ref, logp_ref, psum_ref, msum_ref):
    xp = xp_ref[:]
    xr = xr_ref[:]
    uwg = uwg_ref[:]

    big = (jnp.dot(xp, uwg[:D], preferred_element_type=jnp.float32, precision=_HI)
           + jnp.dot(xr, uwg[D:], preferred_element_type=jnp.float32, precision=_HI)
           + ubg_ref[:])
    u_all = big[:, :R * H]
    logits = big[:, R * H:]

    m = jnp.max(logits, axis=1, keepdims=True)
    e = jnp.exp(logits - m)
    probs = e / jnp.sum(e, axis=1, keepdims=True)

    iota8 = jax.lax.broadcasted_iota(jnp.int32, (BLK, R), 1)
    m1 = jnp.max(logits, axis=1, keepdims=True)
    i1 = jnp.min(jnp.where(logits == m1, iota8, R), axis=1, keepdims=True)
    neg = jnp.where(iota8 == i1, -jnp.inf, logits)
    m2 = jnp.max(neg, axis=1, keepdims=True)
    i2 = jnp.min(jnp.where(neg == m2, iota8, R), axis=1, keepdims=True)

    ew = jnp.exp(m2 - m1)
    denom = 1.0 + ew
    w1 = 1.0 / denom
    w2 = ew / denom
    maskvec = (jnp.where(iota8 == i1, 1.0, 0.0) + jnp.where(iota8 == i2, 1.0, 0.0))

    psum_ref[:] = jnp.sum(probs, axis=0, keepdims=True).reshape(1, 1, R)
    msum_ref[:] = jnp.sum(maskvec, axis=0, keepdims=True).reshape(1, 1, R)

    chunkid = jax.lax.broadcasted_iota(jnp.int32, (BLK, R * H), 1) // H
    v = v_ref[:]
    llm_probs = jnp.zeros((BLK, NL), dtype=jnp.float32)
    for ik, wk in ((i1, w1), (i2, w2)):
        um = jnp.where(chunkid == ik, u_all, 0.0)
        nk = jnp.sqrt(jnp.sum(um * um, axis=1, keepdims=True))
        sck = jax.lax.dot_general(um, v, (((1,), (1,)), ((), ())),
                                  preferred_element_type=jnp.float32,
                                  precision=_HI) / jnp.maximum(nk, 1e-12)
        ms = jnp.max(sck, axis=1, keepdims=True)
        es = jnp.exp(sck - ms)
        ro = es / jnp.sum(es, axis=1, keepdims=True)
        llm_probs += wk * ro

    io_r = jax.lax.broadcasted_iota(jnp.int32, (NL, NL), 0)
    io_c = jax.lax.broadcasted_iota(jnp.int32, (NL, NL), 1)
    tri = (io_r <= io_c).astype(jnp.float32)
    csum = jnp.dot(llm_probs, tri, preferred_element_type=jnp.float32,
                   precision=_HI)
    iota64 = jax.lax.broadcasted_iota(jnp.int32, (BLK, NL), 1)
    gt = csum > rand_ref[:]
    idx = jnp.min(jnp.where(gt, iota64, NL), axis=1, keepdims=True)
    idx = jnp.where(idx == NL, 0, idx)
    sel_ref[:] = idx

    pick = jnp.sum(jnp.where(iota64 == idx, llm_probs, 0.0), axis=1,
                   keepdims=True)
    logp_ref[:] = jnp.log(pick)


def kernel(enhanced_posts_embeddings, selected_reasoning_embeddings,
           llm_embeddings, gate_W, gate_b, U_W, U_b, V_W, V_b, rand_u):
    uwg = jnp.concatenate([U_W.reshape(R * H, 2 * D).T, gate_W.T], axis=1)
    ubg = jnp.concatenate([U_b.reshape(1, R * H), gate_b.reshape(1, R)], axis=1)
    vwt = V_W.reshape(R * H, D).T
    vb = V_b.reshape(1, R * H)

    v_norm = pl.pallas_call(
        _v_kernel,
        out_shape=jax.ShapeDtypeStruct((NL, R * H), jnp.float32),
    )(llm_embeddings, vwt, vb)

    blk = lambda *shape: pl.BlockSpec(shape, lambda i: (0,) * len(shape))
    sel, logp, psum, msum = pl.pallas_call(
        _main_kernel,
        grid=(NBLK,),
        in_specs=[
            pl.BlockSpec((BLK, D), lambda i: (i, 0)),
            pl.BlockSpec((BLK, D), lambda i: (i, 0)),
            blk(2 * D, R * H + R),
            blk(1, R * H + R),
            blk(NL, R * H),
            pl.BlockSpec((BLK, 1), lambda i: (i, 0)),
        ],
        out_specs=[
            pl.BlockSpec((BLK, 1), lambda i: (i, 0)),
            pl.BlockSpec((BLK, 1), lambda i: (i, 0)),
            pl.BlockSpec((1, 1, R), lambda i: (i, 0, 0)),
            pl.BlockSpec((1, 1, R), lambda i: (i, 0, 0)),
        ],
        out_shape=[
            jax.ShapeDtypeStruct((B, 1), jnp.int32),
            jax.ShapeDtypeStruct((B, 1), jnp.float32),
            jax.ShapeDtypeStruct((NBLK, 1, R), jnp.float32),
            jax.ShapeDtypeStruct((NBLK, 1, R), jnp.float32),
        ],
        compiler_params=pltpu.CompilerParams(
            dimension_semantics=("parallel",),
        ),
    )(enhanced_posts_embeddings, selected_reasoning_embeddings, uwg, ubg,
      v_norm, rand_u)

    aux = pl.pallas_call(
        _aux_kernel,
        out_shape=jax.ShapeDtypeStruct((1, 1), jnp.float32),
    )(psum.reshape(NBLK, R), msum.reshape(NBLK, R))

    return sel.reshape(B), logp, aux.reshape(())
